# Initial kernel scaffold; baseline (speedup 1.0000x reference)
import jax
import jax.numpy as jnp
from jax.experimental import pallas as pl
from jax.experimental.pallas import tpu as pltpu

C = 100000
B = 1024
SMOOTH = 0.1
CONF = 1.0 - SMOOTH
SVAL = SMOOTH / (C - 1)

BR = 256
BV = 4096
NR = B // BR
NV = (C + BV - 1) // BV


def _loss_body(t_ref, x_ref, o_ref, m_ref, s_ref, rs_ref, tg_ref):
    r = pl.program_id(0)
    v = pl.program_id(1)
    nv = pl.num_programs(1)

    @pl.when(v == 0)
    def _init():
        m_ref[...] = jnp.full((BR, 1), -jnp.inf, jnp.float32)
        s_ref[...] = jnp.zeros((BR, 1), jnp.float32)
        rs_ref[...] = jnp.zeros((BR, 1), jnp.float32)
        tg_ref[...] = jnp.zeros((BR, 1), jnp.float32)

    x = x_ref[...]
    t = t_ref[...]
    cols = v * BV + jax.lax.broadcasted_iota(jnp.int32, (BR, BV), 1)

    def update(xm, xz):
        m_old = m_ref[...]
        bmax = jnp.max(xm, axis=1, keepdims=True)
        m_new = jnp.maximum(m_old, bmax)
        e = jnp.exp(xm - m_new)
        s_ref[...] = s_ref[...] * jnp.exp(m_old - m_new) + jnp.sum(
            e, axis=1, keepdims=True)
        m_ref[...] = m_new
        rs_ref[...] += jnp.sum(xz, axis=1, keepdims=True)
        tg_ref[...] += jnp.sum(jnp.where(cols == t, xz, 0.0), axis=1,
                               keepdims=True)

    @pl.when(v < nv - 1)
    def _full():
        update(x, x)

    @pl.when(v == nv - 1)
    def _last():
        valid = cols < C
        update(jnp.where(valid, x, -jnp.inf), jnp.where(valid, x, 0.0))
        lse = m_ref[...] + jnp.log(s_ref[...])
        loss = -(SVAL * (rs_ref[...] - C * lse)
                 + (CONF - SVAL) * (tg_ref[...] - lse))
        part = jnp.sum(loss) / B

        @pl.when(r == 0)
        def _():
            o_ref[0, 0] = part

        @pl.when(r > 0)
        def _():
            o_ref[0, 0] += part


def kernel(inputs, targets):
    t2 = targets.reshape(B, 1)
    out = pl.pallas_call(
        _loss_body,
        grid=(NR, NV),
        in_specs=[
            pl.BlockSpec((BR, 1), lambda r, v: (r, 0)),
            pl.BlockSpec((BR, BV), lambda r, v: (r, v)),
        ],
        out_specs=pl.BlockSpec((1, 1), lambda r, v: (0, 0)),
        out_shape=jax.ShapeDtypeStruct((1, 1), jnp.float32),
        scratch_shapes=[pltpu.VMEM((BR, 1), jnp.float32) for _ in range(4)],
    )(t2, inputs)
    return out[0, 0]


# speedup vs baseline: 2.0050x; 2.0050x over previous
import jax
import jax.numpy as jnp
from jax.experimental import pallas as pl
from jax.experimental.pallas import tpu as pltpu

C = 100000
B = 1024
SMOOTH = 0.1
CONF = 1.0 - SMOOTH
SVAL = SMOOTH / (C - 1)

BR = 256
BV = 4096
NR = B // BR
NV = (C + BV - 1) // BV


def _loss_body(t_ref, x_ref, o_ref, m_ref, s_ref, rs_ref, tg_ref):
    r = pl.program_id(0)
    v = pl.program_id(1)
    nv = pl.num_programs(1)

    @pl.when(v == 0)
    def _init():
        m_ref[...] = jnp.full((BR, 1), -jnp.inf, jnp.float32)
        s_ref[...] = jnp.zeros((BR, 1), jnp.float32)
        rs_ref[...] = jnp.zeros((BR, 1), jnp.float32)
        tg_ref[...] = jnp.zeros((BR, 1), jnp.float32)

    x = x_ref[...]
    t = t_ref[...]
    cols = v * BV + jax.lax.broadcasted_iota(jnp.int32, (BR, BV), 1)

    def update(xm, xz):
        m_old = m_ref[...]
        bmax = jnp.max(xm, axis=1, keepdims=True)
        m_new = jnp.maximum(m_old, bmax)
        e = jnp.exp(xm - m_new)
        s_ref[...] = s_ref[...] * jnp.exp(m_old - m_new) + jnp.sum(
            e, axis=1, keepdims=True)
        m_ref[...] = m_new
        rs_ref[...] += jnp.sum(xz, axis=1, keepdims=True)
        tg_ref[...] += jnp.sum(jnp.where(cols == t, xz, 0.0), axis=1,
                               keepdims=True)

    @pl.when(v < nv - 1)
    def _full():
        update(x, x)

    @pl.when(v == nv - 1)
    def _last():
        valid = cols < C
        update(jnp.where(valid, x, -jnp.inf), jnp.where(valid, x, 0.0))
        lse = m_ref[...] + jnp.log(s_ref[...])
        loss = -(SVAL * (rs_ref[...] - C * lse)
                 + (CONF - SVAL) * (tg_ref[...] - lse))
        part = jnp.reshape(jnp.sum(loss) / B, (1, 1))

        @pl.when(r == 0)
        def _():
            o_ref[...] = part

        @pl.when(r > 0)
        def _():
            o_ref[...] = o_ref[...] + part


def kernel(inputs, targets):
    t2 = targets.reshape(B, 1)
    out = pl.pallas_call(
        _loss_body,
        grid=(NR, NV),
        in_specs=[
            pl.BlockSpec((BR, 1), lambda r, v: (r, 0)),
            pl.BlockSpec((BR, BV), lambda r, v: (r, v)),
        ],
        out_specs=pl.BlockSpec((1, 1), lambda r, v: (0, 0)),
        out_shape=jax.ShapeDtypeStruct((1, 1), jnp.float32),
        scratch_shapes=[pltpu.VMEM((BR, 1), jnp.float32) for _ in range(4)],
    )(t2, inputs)
    return out[0, 0]
